# Initial kernel scaffold; baseline (speedup 1.0000x reference)
import jax
import jax.numpy as jnp
from jax.experimental import pallas as pl


def kernel(x, edge_index, edge_weight, x_0, W, b):
    raise NotImplementedError("write your pallas kernel here")


# speedup vs baseline: 6.3748x; 6.3748x over previous
import functools

import jax
import jax.numpy as jnp
from jax import lax
from jax.experimental import pallas as pl
from jax.experimental.pallas import tpu as pltpu
from jax.experimental.pallas import tpu_sc as plsc

N = 10000
E = 320000
D = 128
GAMMA = 0.9

NC = 2
NS = 16
NW = NC * NS
EPT = E // NW
SUP = 2000
NSUP = EPT // SUP
CHUNK = 80
NCHUNK = SUP // CHUNK
GRP = CHUNK // 16
NPAD = 10240
RPT = NPAD // NS
WBLK = 64


def _lane_bcast(v16, e):
    idx = jnp.full((16, 1), e, dtype=jnp.int32)
    dn = lax.GatherDimensionNumbers(
        offset_dims=(), collapsed_slice_dims=(0,), start_index_map=(0,))
    return lax.gather(v16, idx, dn, (1,),
                      mode=lax.GatherScatterMode.PROMISE_IN_BOUNDS)


def _sc_spmm(src, dst, ew, x, zeros_blk):
    mesh = plsc.VectorSubcoreMesh(core_axis_name="c", subcore_axis_name="s",
                                  num_cores=NC, num_subcores=NS)

    @functools.partial(
        pl.kernel,
        out_type=jax.ShapeDtypeStruct((NC, NPAD, D), jnp.float32),
        mesh=mesh,
        scratch_types=[
            pltpu.VMEM((SUP,), jnp.int32),
            pltpu.VMEM((SUP,), jnp.int32),
            pltpu.VMEM((SUP,), jnp.float32),
            pltpu.VMEM((CHUNK, D), jnp.float32),
            pltpu.VMEM((CHUNK,), jnp.int32),
            pltpu.VMEM((WBLK, D), jnp.float32),
            pltpu.SemaphoreType.DMA,
            pltpu.VMEM_SHARED((NPAD, D), jnp.float32),
        ],
    )
    def spmm(src_hbm, dst_hbm, ew_hbm, x_hbm, z_hbm, part_hbm,
             src_v, dst_v, ew_v, rows, dcur, bounce, gsem, acc):
        cid = lax.axis_index("c")
        sid = lax.axis_index("s")
        wid = sid * NC + cid

        pltpu.sync_copy(z_hbm, bounce)
        for k in range(RPT // WBLK):
            pltpu.sync_copy(bounce, acc.at[pl.ds(sid * RPT + k * WBLK, WBLK)])
        plsc.subcore_barrier()

        ebase = wid * EPT

        def sup_body(u, carry):
            sbase = ebase + u * SUP
            pltpu.sync_copy(src_hbm.at[pl.ds(sbase, SUP)], src_v)
            pltpu.sync_copy(dst_hbm.at[pl.ds(sbase, SUP)], dst_v)
            pltpu.sync_copy(ew_hbm.at[pl.ds(sbase, SUP)], ew_v)

            def chunk_body(c, carry2):
                cbase = c * CHUNK
                pltpu.async_copy(
                    x_hbm.at[src_v.at[pl.ds(cbase, CHUNK)]], rows, gsem).wait()
                for g in range(GRP):
                    w16 = ew_v[pl.ds(cbase + g * 16, 16)]
                    dcur[pl.ds(g * 16, 16)] = dst_v[pl.ds(cbase + g * 16, 16)]
                    for e in range(16):
                        splat = _lane_bcast(w16, e)
                        r = g * 16 + e
                        for j in range(D // 16):
                            rows[r, pl.ds(j * 16, 16)] = (
                                rows[r, pl.ds(j * 16, 16)] * splat)
                pltpu.sync_copy(rows, acc.at[dcur], add=True)
                return carry2

            lax.fori_loop(0, NCHUNK, chunk_body, 0)
            return carry

        lax.fori_loop(0, NSUP, sup_body, 0)
        plsc.subcore_barrier()

        for k in range(RPT // WBLK):
            r0 = sid * RPT + k * WBLK
            pltpu.sync_copy(acc.at[pl.ds(r0, WBLK)], bounce)
            pltpu.sync_copy(bounce, part_hbm.at[cid, pl.ds(r0, WBLK)])

    return spmm(src, dst, ew, x, zeros_blk)


def _combine_body(p_ref, x0_ref, b_ref, o_ref):
    o_ref[...] = ((p_ref[0] + p_ref[1]) * GAMMA
                  + x0_ref[...] + b_ref[...])


def _combine(part, x_0, b):
    blk = 1000
    return pl.pallas_call(
        _combine_body,
        grid=(N // blk,),
        in_specs=[
            pl.BlockSpec((NC, blk, D), lambda i: (0, i, 0)),
            pl.BlockSpec((blk, D), lambda i: (i, 0)),
            pl.BlockSpec((1, D), lambda i: (0, 0)),
        ],
        out_specs=pl.BlockSpec((blk, D), lambda i: (i, 0)),
        out_shape=jax.ShapeDtypeStruct((N, D), jnp.float32),
    )(part, x_0, b)


def kernel(x, edge_index, edge_weight, x_0, W, b):
    del W
    src = edge_index[0]
    dst = edge_index[1]
    zeros_blk = jnp.zeros((WBLK, D), jnp.float32)
    part = _sc_spmm(src, dst, edge_weight, x, zeros_blk)
    return _combine(part, x_0, b.reshape(1, D))
